# Initial kernel scaffold; baseline (speedup 1.0000x reference)
import jax
import jax.numpy as jnp
from jax.experimental import pallas as pl


def kernel(f_BChw, codebook, phi_w, phi_b):
    raise NotImplementedError("write your pallas kernel here")


# speedup vs baseline: 1.1913x; 1.1913x over previous
import functools

import jax
import jax.numpy as jnp
import numpy as np
from jax import lax
from jax.experimental import pallas as pl
from jax.experimental.pallas import tpu as pltpu
from jax.experimental.pallas import tpu_sc as plsc

_PNS = (1, 2, 4, 8, 16, 32, 64, 128, 256, 512)
_VOCAB = 8192
_C = 256
_B = 16
_H = 512
_BETA = 0.25
_SHARE = 4
_NSC = 10
_CB = 1024
_NKB = _VOCAB // _CB


def _cubic_w_np(t, a=-0.75):
    at = np.abs(t)
    w1 = (a + 2.0) * at ** 3 - (a + 3.0) * at ** 2 + 1.0
    w2 = a * at ** 3 - 5.0 * a * at ** 2 + 8.0 * a * at - 4.0 * a
    return np.where(at <= 1.0, w1, np.where(at < 2.0, w2, np.zeros_like(at)))


@functools.lru_cache(maxsize=None)
def _upsample_matrix(pn: int, out_h: int) -> np.ndarray:
    scale = pn / out_h
    i = np.arange(out_h, dtype=np.float32)
    src = (i + 0.5) * scale - 0.5
    i0 = np.floor(src).astype(np.int32)
    u = np.zeros((out_h, pn), dtype=np.float32)
    for t in range(4):
        tap = i0 - 1 + t
        w = _cubic_w_np((src - tap).astype(np.float32))
        tap_c = np.clip(tap, 0, pn - 1)
        for y in range(out_h):
            u[y, tap_c[y]] += w[y]
    return u


@functools.lru_cache(maxsize=None)
def _phi_share(si: int) -> int:
    ticks = np.linspace(1.0 / 3.0 / _SHARE, 1.0 - 1.0 / 3.0 / _SHARE, _SHARE)
    return int(np.argmin(np.abs(ticks - si / (_NSC - 1))))


def _prologue_kernel(cb_ref, f_ref, cbn_ref, ds_ref):
    cb = cb_ref[...]
    nrm = jnp.sqrt(jnp.sum(cb * cb, axis=1))
    cbn_ref[...] = cb / jnp.maximum(nrm, 1e-12)[:, None]
    ds_ref[0, 0, :] = jnp.mean(f_ref[0], axis=0)


def _prologue(codebook, f_r):
    return pl.pallas_call(
        _prologue_kernel,
        grid=(_B,),
        in_specs=[
            pl.BlockSpec((_VOCAB // _B, _C), lambda i: (i, 0)),
            pl.BlockSpec((1, _H, _C), lambda i: (i, 0, 0)),
        ],
        out_specs=[
            pl.BlockSpec((_VOCAB // _B, _C), lambda i: (i, 0)),
            pl.BlockSpec((1, 1, _C), lambda i: (i, 0, 0)),
        ],
        out_shape=[
            jax.ShapeDtypeStruct((_VOCAB, _C), jnp.float32),
            jax.ShapeDtypeStruct((_B, 1, _C), jnp.float32),
        ],
    )(codebook, f_r)


def _argmax_kernel(q_ref, cb_ref, idx_ref, m_ref, a_ref):
    k = pl.program_id(1)
    q = q_ref[...]
    qn = q / jnp.maximum(jnp.sqrt(jnp.sum(q * q, axis=1)), 1e-12)[:, None]
    cb = cb_ref[...]
    s = lax.dot_general(qn.astype(jnp.bfloat16), cb.astype(jnp.bfloat16),
                        (((1,), (1,)), ((), ())),
                        preferred_element_type=jnp.float32)
    lmax = jnp.max(s, axis=1)
    larg = jnp.argmax(s, axis=1).astype(jnp.int32) + k * _CB

    @pl.when(k == 0)
    def _():
        m_ref[0, :] = lmax
        a_ref[0, :] = larg

    @pl.when(k > 0)
    def _():
        m = m_ref[0, :]
        better = lmax > m
        m_ref[0, :] = jnp.where(better, lmax, m)
        a_ref[0, :] = jnp.where(better, larg, a_ref[0, :])

    @pl.when(k == _NKB - 1)
    def _():
        idx_ref[0, 0, :] = a_ref[0, :]


def _argmax(q, cbn):
    n = q.shape[0]
    rb = min(n, 512)
    nrb = n // rb
    idx = pl.pallas_call(
        _argmax_kernel,
        grid=(nrb, _NKB),
        in_specs=[
            pl.BlockSpec((rb, _C), lambda i, k: (i, 0)),
            pl.BlockSpec((_CB, _C), lambda i, k: (k, 0)),
        ],
        out_specs=pl.BlockSpec((1, 1, rb), lambda i, k: (i, 0, 0)),
        out_shape=jax.ShapeDtypeStruct((nrb, 1, rb), jnp.int32),
        scratch_shapes=[
            pltpu.VMEM((1, rb), jnp.float32),
            pltpu.VMEM((1, rb), jnp.int32),
        ],
    )(q, cbn)
    return idx


def _sc_gather(codebook, idx_pad):
    info = plsc.get_sparse_core_info()
    nw = info.num_cores * info.num_subcores
    npad = idx_pad.shape[0]
    b_per_w = npad // nw
    mesh = plsc.VectorSubcoreMesh(core_axis_name="c", subcore_axis_name="s")

    @functools.partial(
        pl.kernel, mesh=mesh,
        out_type=jax.ShapeDtypeStruct((npad, _C), jnp.float32),
        scratch_types=[
            pltpu.VMEM((b_per_w,), jnp.int32),
            pltpu.VMEM((b_per_w, _C), jnp.float32),
            pltpu.SemaphoreType.DMA,
        ],
    )
    def k(table_hbm, idx_hbm, out_hbm, idx_v, rows_v, sem):
        wid = lax.axis_index("s") * info.num_cores + lax.axis_index("c")
        base = wid * b_per_w
        pltpu.sync_copy(idx_hbm.at[pl.ds(base, b_per_w)], idx_v)
        pltpu.async_copy(table_hbm.at[idx_v], rows_v, sem).wait()
        pltpu.sync_copy(rows_v, out_hbm.at[pl.ds(base, b_per_w)])

    return k(codebook, idx_pad)


def _update_kernel(si, pn, pn_next, g_ref, u_ref, w_ref, b_ref, rest_ref,
                   f_ref, rest_out, ds_out, ss_out, fhat_out):
    b = pl.program_id(0)
    g = g_ref[0]
    if si != _NSC - 1:
        gp = lax.dot_general(u_ref[...], g, (((1,), (0,)), ((), ())),
                             preferred_element_type=jnp.float32,
                             precision=lax.Precision.HIGHEST)
    else:
        gp = g
    gpb = gp.astype(jnp.bfloat16)
    zrow = jnp.zeros((1, _C), jnp.bfloat16)
    sd = jnp.concatenate([zrow, gpb[:-1, :]], axis=0)
    su = jnp.concatenate([gpb[1:, :], zrow], axis=0)
    mm = lambda x, w: lax.dot_general(
        x, w, (((1,), (0,)), ((), ())), preferred_element_type=jnp.float32)
    w = w_ref[...].astype(jnp.bfloat16)
    y2 = mm(sd, w[0]) + mm(gpb, w[1]) + mm(su, w[2])
    h = 0.5 * gp + 0.5 * (y2 + b_ref[0, :][None, :])
    rnew = rest_ref[0] - h
    if rest_out is not None:
        rest_out[0] = rnew
    if si != _NSC - 1:
        r_next = _H // pn_next
        ds_out[0] = jnp.mean(rnew.reshape(pn_next, r_next, _C), axis=1)
    else:
        fhat_out[0] = f_ref[0] - rnew

    @pl.when(b == 0)
    def _():
        ss_out[0, 0] = 0.0

    ss_out[0, 0] += jnp.sum(rnew * rnew)


def _update(si, g, rest, f_r, u_mat, w3, bias):
    pn = _PNS[si]
    last = si == _NSC - 1
    pn_next = _PNS[si + 1] if not last else 0

    in_specs = [
        pl.BlockSpec((1, pn, _C), lambda b: (b, 0, 0)),
        pl.BlockSpec((_H, pn), lambda b: (0, 0)),
        pl.BlockSpec((3, _C, _C), lambda b: (0, 0, 0)),
        pl.BlockSpec((1, _C), lambda b: (0, 0)),
        pl.BlockSpec((1, _H, _C), lambda b: (b, 0, 0)),
        pl.BlockSpec((1, _H, _C), lambda b: (b, 0, 0)),
    ]
    out_specs = []
    out_shape = []
    if not last:
        out_specs.append(pl.BlockSpec((1, _H, _C), lambda b: (b, 0, 0)))
        out_shape.append(jax.ShapeDtypeStruct((_B, _H, _C), jnp.float32))
        out_specs.append(pl.BlockSpec((1, pn_next, _C), lambda b: (b, 0, 0)))
        out_shape.append(jax.ShapeDtypeStruct((_B, pn_next, _C), jnp.float32))
    out_specs.append(pl.BlockSpec((1, 1), lambda b: (0, 0),
                                  memory_space=pltpu.SMEM))
    out_shape.append(jax.ShapeDtypeStruct((1, 1), jnp.float32))
    if last:
        out_specs.append(pl.BlockSpec((1, _H, _C), lambda b: (b, 0, 0)))
        out_shape.append(jax.ShapeDtypeStruct((_B, _H, _C), jnp.float32))

    def body(g_ref, u_ref, w_ref, b_ref, rest_ref, f_ref, *outs):
        if last:
            ss_out, fhat_out = outs
            _update_kernel(si, pn, pn_next, g_ref, u_ref, w_ref, b_ref,
                           rest_ref, f_ref, None, None, ss_out, fhat_out)
        else:
            rest_out, ds_out, ss_out = outs
            _update_kernel(si, pn, pn_next, g_ref, u_ref, w_ref, b_ref,
                           rest_ref, f_ref, rest_out, ds_out, ss_out, None)

    outs = pl.pallas_call(
        body,
        grid=(_B,),
        in_specs=in_specs,
        out_specs=out_specs,
        out_shape=out_shape,
    )(g, u_mat, w3, bias, rest, f_r)
    return outs


def kernel(f_BChw, codebook, phi_w, phi_b):
    f_r = jnp.transpose(f_BChw[..., 0], (0, 2, 1))

    info = plsc.get_sparse_core_info()
    nw_align = 8 * info.num_cores * info.num_subcores

    cbn, ds0 = _prologue(codebook, f_r)

    w3s, biases = [], []
    for si in range(_NSC):
        k = _phi_share(si)
        w3s.append(jnp.transpose(phi_w[k, :, :, :, 1], (2, 1, 0)))
        biases.append(phi_b[k].reshape(1, _C))

    rest = f_r
    ds = ds0.reshape(_B, _C)
    ss_list = []
    fhat = None
    for si, pn in enumerate(_PNS):
        n = _B * pn
        idx = _argmax(ds.reshape(n, _C), cbn).reshape(n)
        npad = max(n, nw_align)
        if npad > n:
            idx = jnp.concatenate([idx, jnp.zeros((npad - n,), jnp.int32)])
        g = _sc_gather(codebook, idx)[:n].reshape(_B, pn, _C)
        if si != _NSC - 1:
            u_mat = jnp.asarray(_upsample_matrix(pn, _H))
        else:
            u_mat = jnp.zeros((_H, pn), jnp.float32)
        outs = _update(si, g, rest, f_r, u_mat, w3s[si], biases[si])
        if si != _NSC - 1:
            rest, ds_next, ss = outs
            ds = ds_next.reshape(_B * _PNS[si + 1], _C)
        else:
            ss, fhat = outs
        ss_list.append(ss[0, 0])

    numel = _B * _H * _C
    loss = (1.0 + _BETA) / _NSC * jnp.sum(jnp.stack(ss_list)) / numel
    f_hat_out = jnp.transpose(fhat, (0, 2, 1))[..., None]
    return (f_hat_out, loss)


# speedup vs baseline: 1.2710x; 1.0669x over previous
import functools

import jax
import jax.numpy as jnp
import numpy as np
from jax import lax
from jax.experimental import pallas as pl
from jax.experimental.pallas import tpu as pltpu
from jax.experimental.pallas import tpu_sc as plsc

_PNS = (1, 2, 4, 8, 16, 32, 64, 128, 256, 512)
_VOCAB = 8192
_C = 256
_B = 16
_H = 512
_BETA = 0.25
_SHARE = 4
_NSC = 10


def _cubic_w_np(t, a=-0.75):
    at = np.abs(t)
    w1 = (a + 2.0) * at ** 3 - (a + 3.0) * at ** 2 + 1.0
    w2 = a * at ** 3 - 5.0 * a * at ** 2 + 8.0 * a * at - 4.0 * a
    return np.where(at <= 1.0, w1, np.where(at < 2.0, w2, np.zeros_like(at)))


@functools.lru_cache(maxsize=None)
def _upsample_matrix(pn: int, out_h: int) -> np.ndarray:
    scale = pn / out_h
    i = np.arange(out_h, dtype=np.float32)
    src = (i + 0.5) * scale - 0.5
    i0 = np.floor(src).astype(np.int32)
    u = np.zeros((out_h, pn), dtype=np.float32)
    for t in range(4):
        tap = i0 - 1 + t
        w = _cubic_w_np((src - tap).astype(np.float32))
        tap_c = np.clip(tap, 0, pn - 1)
        for y in range(out_h):
            u[y, tap_c[y]] += w[y]
    return u


@functools.lru_cache(maxsize=None)
def _phi_share(si: int) -> int:
    ticks = np.linspace(1.0 / 3.0 / _SHARE, 1.0 - 1.0 / 3.0 / _SHARE, _SHARE)
    return int(np.argmin(np.abs(ticks - si / (_NSC - 1))))


def _code_tile(pn_next: int) -> int:
    return _VOCAB if pn_next <= 64 else 2048


def _prologue_kernel(cb_ref, f_ref, cbn_ref, ds_ref):
    cb = cb_ref[...]
    nrm = jnp.sqrt(jnp.sum(cb * cb, axis=1))
    cbn_ref[...] = cb / jnp.maximum(nrm, 1e-12)[:, None]
    ds_ref[0, 0, :] = jnp.mean(f_ref[0], axis=0)


def _prologue(codebook, f_r):
    return pl.pallas_call(
        _prologue_kernel,
        grid=(_B,),
        in_specs=[
            pl.BlockSpec((_VOCAB // _B, _C), lambda i: (i, 0)),
            pl.BlockSpec((1, _H, _C), lambda i: (i, 0, 0)),
        ],
        out_specs=[
            pl.BlockSpec((_VOCAB // _B, _C), lambda i: (i, 0)),
            pl.BlockSpec((1, 1, _C), lambda i: (i, 0, 0)),
        ],
        out_shape=[
            jax.ShapeDtypeStruct((_VOCAB, _C), jnp.float32),
            jax.ShapeDtypeStruct((_B, 1, _C), jnp.float32),
        ],
    )(codebook, f_r)


def _normalize_rows(q):
    return q / jnp.maximum(jnp.sqrt(jnp.sum(q * q, axis=1)), 1e-12)[:, None]


def _score(qn, cbn):
    return lax.dot_general(qn.astype(jnp.bfloat16), cbn.astype(jnp.bfloat16),
                           (((1,), (1,)), ((), ())),
                           preferred_element_type=jnp.float32)


def _argmax0_kernel(q_ref, cbn_ref, idx_ref):
    qn = _normalize_rows(q_ref[...])
    s = _score(qn, cbn_ref[...])
    idx_ref[0, 0, :] = jnp.argmax(s, axis=1).astype(jnp.int32)


def _argmax0(ds0, cbn):
    return pl.pallas_call(
        _argmax0_kernel,
        out_specs=pl.BlockSpec((1, 1, _B), lambda: (0, 0, 0)),
        out_shape=jax.ShapeDtypeStruct((1, 1, _B), jnp.int32),
    )(ds0, cbn)


def _sc_gather(codebook, idx):
    info = plsc.get_sparse_core_info()
    nw = info.num_cores * info.num_subcores
    n = idx.shape[0]
    nw_used = min(nw, n // 8)
    b_per_w = n // nw_used
    mesh = plsc.VectorSubcoreMesh(core_axis_name="c", subcore_axis_name="s")

    @functools.partial(
        pl.kernel, mesh=mesh,
        out_type=jax.ShapeDtypeStruct((n, _C), jnp.float32),
        scratch_types=[
            pltpu.VMEM((b_per_w,), jnp.int32),
            pltpu.VMEM((b_per_w, _C), jnp.float32),
            pltpu.SemaphoreType.DMA,
        ],
    )
    def k(table_hbm, idx_hbm, out_hbm, idx_v, rows_v, sem):
        wid = lax.axis_index("s") * info.num_cores + lax.axis_index("c")

        @pl.when(wid < nw_used)
        def _():
            base = wid * b_per_w
            pltpu.sync_copy(idx_hbm.at[pl.ds(base, b_per_w)], idx_v)
            pltpu.async_copy(table_hbm.at[idx_v], rows_v, sem).wait()
            pltpu.sync_copy(rows_v, out_hbm.at[pl.ds(base, b_per_w)])

    return k(codebook, idx)


def _phi_update(si, g, u_ref, w_ref, b_ref, rest_blk):
    if si != _NSC - 1:
        gp = lax.dot_general(u_ref[...], g, (((1,), (0,)), ((), ())),
                             preferred_element_type=jnp.float32,
                             precision=lax.Precision.HIGHEST)
    else:
        gp = g
    gpb = gp.astype(jnp.bfloat16)
    zrow = jnp.zeros((1, _C), jnp.bfloat16)
    sd = jnp.concatenate([zrow, gpb[:-1, :]], axis=0)
    su = jnp.concatenate([gpb[1:, :], zrow], axis=0)
    mm = lambda x, w: lax.dot_general(
        x, w, (((1,), (0,)), ((), ())), preferred_element_type=jnp.float32)
    w = w_ref[...].astype(jnp.bfloat16)
    y2 = mm(sd, w[0]) + mm(gpb, w[1]) + mm(su, w[2])
    h = 0.5 * gp + 0.5 * (y2 + b_ref[0, :][None, :])
    return rest_blk - h


def _merged_kernel(si, pn, pn_next, nkb, g_ref, u_ref, w_ref, b_ref,
                   rest_ref, cbn_ref, rest_out, idx_out, ss_out,
                   ds_s, m_s, a_s):
    b = pl.program_id(0)
    k = pl.program_id(1)
    cbs = _VOCAB // nkb

    @pl.when(k == 0)
    def _():
        rnew = _phi_update(si, g_ref[0], u_ref, w_ref, b_ref, rest_ref[0])
        rest_out[0] = rnew
        r_next = _H // pn_next
        if r_next > 1:
            dsv = jnp.mean(rnew.reshape(pn_next, r_next, _C), axis=1)
        else:
            dsv = rnew
        ds_s[...] = _normalize_rows(dsv)
        ssq = jnp.sum(rnew * rnew)

        @pl.when(b == 0)
        def _():
            ss_out[0, 0] = ssq

        @pl.when(b > 0)
        def _():
            ss_out[0, 0] += ssq

    s = _score(ds_s[...], cbn_ref[...])
    lmax = jnp.max(s, axis=1)
    larg = jnp.argmax(s, axis=1).astype(jnp.int32) + k * cbs

    @pl.when(k == 0)
    def _():
        m_s[0, :] = lmax
        a_s[0, :] = larg

    @pl.when(k > 0)
    def _():
        m = m_s[0, :]
        better = lmax > m
        m_s[0, :] = jnp.where(better, lmax, m)
        a_s[0, :] = jnp.where(better, larg, a_s[0, :])

    @pl.when(k == nkb - 1)
    def _():
        idx_ref = a_s[0, :]
        idx_out[0, 0, :] = idx_ref


def _merged(si, g, rest, cbn, u_mat, w3, bias):
    pn = _PNS[si]
    pn_next = _PNS[si + 1]
    cbs = _code_tile(pn_next)
    nkb = _VOCAB // cbs

    body = functools.partial(_merged_kernel, si, pn, pn_next, nkb)
    return pl.pallas_call(
        body,
        grid=(_B, nkb),
        in_specs=[
            pl.BlockSpec((1, pn, _C), lambda b, k: (b, 0, 0)),
            pl.BlockSpec((_H, pn), lambda b, k: (0, 0)),
            pl.BlockSpec((3, _C, _C), lambda b, k: (0, 0, 0)),
            pl.BlockSpec((1, _C), lambda b, k: (0, 0)),
            pl.BlockSpec((1, _H, _C), lambda b, k: (b, 0, 0)),
            pl.BlockSpec((cbs, _C), lambda b, k: (k, 0)),
        ],
        out_specs=[
            pl.BlockSpec((1, _H, _C), lambda b, k: (b, 0, 0)),
            pl.BlockSpec((1, 1, pn_next), lambda b, k: (b, 0, 0)),
            pl.BlockSpec((1, 1), lambda b, k: (0, 0), memory_space=pltpu.SMEM),
        ],
        out_shape=[
            jax.ShapeDtypeStruct((_B, _H, _C), jnp.float32),
            jax.ShapeDtypeStruct((_B, 1, pn_next), jnp.int32),
            jax.ShapeDtypeStruct((1, 1), jnp.float32),
        ],
        scratch_shapes=[
            pltpu.VMEM((pn_next, _C), jnp.float32),
            pltpu.VMEM((1, pn_next), jnp.float32),
            pltpu.VMEM((1, pn_next), jnp.int32),
        ],
    )(g.reshape(_B, pn, _C), u_mat, w3, bias, rest, cbn)


def _last_kernel(g_ref, w_ref, b_ref, rest_ref, f_ref, ss_out, fhat_out):
    b = pl.program_id(0)
    rnew = _phi_update(_NSC - 1, g_ref[0], None, w_ref, b_ref, rest_ref[0])
    fhat_out[0] = f_ref[0] - rnew
    ssq = jnp.sum(rnew * rnew)

    @pl.when(b == 0)
    def _():
        ss_out[0, 0] = ssq

    @pl.when(b > 0)
    def _():
        ss_out[0, 0] += ssq


def _update_last(g, rest, f_r, w3, bias):
    pn = _PNS[-1]
    return pl.pallas_call(
        _last_kernel,
        grid=(_B,),
        in_specs=[
            pl.BlockSpec((1, pn, _C), lambda b: (b, 0, 0)),
            pl.BlockSpec((3, _C, _C), lambda b: (0, 0, 0)),
            pl.BlockSpec((1, _C), lambda b: (0, 0)),
            pl.BlockSpec((1, _H, _C), lambda b: (b, 0, 0)),
            pl.BlockSpec((1, _H, _C), lambda b: (b, 0, 0)),
        ],
        out_specs=[
            pl.BlockSpec((1, 1), lambda b: (0, 0), memory_space=pltpu.SMEM),
            pl.BlockSpec((1, _H, _C), lambda b: (b, 0, 0)),
        ],
        out_shape=[
            jax.ShapeDtypeStruct((1, 1), jnp.float32),
            jax.ShapeDtypeStruct((_B, _H, _C), jnp.float32),
        ],
    )(g.reshape(_B, pn, _C), w3, bias, rest, f_r)


def kernel(f_BChw, codebook, phi_w, phi_b):
    f_r = jnp.transpose(f_BChw[..., 0], (0, 2, 1))

    cbn, ds0 = _prologue(codebook, f_r)

    w3s, biases = [], []
    for si in range(_NSC):
        k = _phi_share(si)
        w3s.append(jnp.transpose(phi_w[k, :, :, :, 1], (2, 1, 0)))
        biases.append(phi_b[k].reshape(1, _C))

    rest = f_r
    idx = _argmax0(ds0.reshape(_B, _C), cbn).reshape(_B)
    ss_list = []
    for si in range(_NSC - 1):
        g = _sc_gather(codebook, idx)
        u_mat = jnp.asarray(_upsample_matrix(_PNS[si], _H))
        rest, idx3, ss = _merged(si, g, rest, cbn, u_mat, w3s[si], biases[si])
        idx = idx3.reshape(_B * _PNS[si + 1])
        ss_list.append(ss[0, 0])

    g = _sc_gather(codebook, idx)
    ss9, fhat = _update_last(g, rest, f_r, w3s[-1], biases[-1])
    ss_list.append(ss9[0, 0])

    numel = _B * _H * _C
    loss = (1.0 + _BETA) / _NSC * jnp.sum(jnp.stack(ss_list)) / numel
    f_hat_out = jnp.transpose(fhat, (0, 2, 1))[..., None]
    return (f_hat_out, loss)


# speedup vs baseline: 1.4166x; 1.1145x over previous
import functools

import jax
import jax.numpy as jnp
import numpy as np
from jax import lax
from jax.experimental import pallas as pl
from jax.experimental.pallas import tpu as pltpu
from jax.experimental.pallas import tpu_sc as plsc

_PNS = (1, 2, 4, 8, 16, 32, 64, 128, 256, 512)
_VOCAB = 8192
_C = 256
_B = 16
_H = 512
_BETA = 0.25
_SHARE = 4
_NSC = 10


def _cubic_w_np(t, a=-0.75):
    at = np.abs(t)
    w1 = (a + 2.0) * at ** 3 - (a + 3.0) * at ** 2 + 1.0
    w2 = a * at ** 3 - 5.0 * a * at ** 2 + 8.0 * a * at - 4.0 * a
    return np.where(at <= 1.0, w1, np.where(at < 2.0, w2, np.zeros_like(at)))


@functools.lru_cache(maxsize=None)
def _upsample_weights(pn: int, out_h: int) -> np.ndarray:
    scale = pn / out_h
    i = np.arange(out_h, dtype=np.float32)
    src = (i + 0.5) * scale - 0.5
    i0 = np.floor(src).astype(np.int32)
    w = np.zeros((out_h, 4), dtype=np.float32)
    for t in range(4):
        tap = i0 - 1 + t
        w[:, t] = _cubic_w_np((src - tap).astype(np.float32))
    return w


@functools.lru_cache(maxsize=None)
def _phi_share(si: int) -> int:
    ticks = np.linspace(1.0 / 3.0 / _SHARE, 1.0 - 1.0 / 3.0 / _SHARE, _SHARE)
    return int(np.argmin(np.abs(ticks - si / (_NSC - 1))))


def _code_tile(pn_next: int) -> int:
    return _VOCAB if pn_next <= 64 else 2048


def _prologue_kernel(cb_ref, f_ref, cbn_ref, ds_ref, fr_ref):
    cb = cb_ref[...]
    nrm = jnp.sqrt(jnp.sum(cb * cb, axis=1))
    cbn_ref[...] = cb / jnp.maximum(nrm, 1e-12)[:, None]
    fb = f_ref[0]
    ds_ref[0, 0, :] = jnp.mean(fb, axis=1)
    fr_ref[0] = jnp.transpose(fb)


def _prologue(codebook, f_nat):
    return pl.pallas_call(
        _prologue_kernel,
        grid=(_B,),
        in_specs=[
            pl.BlockSpec((_VOCAB // _B, _C), lambda i: (i, 0)),
            pl.BlockSpec((1, _C, _H), lambda i: (i, 0, 0)),
        ],
        out_specs=[
            pl.BlockSpec((_VOCAB // _B, _C), lambda i: (i, 0)),
            pl.BlockSpec((1, 1, _C), lambda i: (i, 0, 0)),
            pl.BlockSpec((1, _H, _C), lambda i: (i, 0, 0)),
        ],
        out_shape=[
            jax.ShapeDtypeStruct((_VOCAB, _C), jnp.float32),
            jax.ShapeDtypeStruct((_B, 1, _C), jnp.float32),
            jax.ShapeDtypeStruct((_B, _H, _C), jnp.float32),
        ],
    )(codebook, f_nat)


def _normalize_rows(q):
    return q / jnp.maximum(jnp.sqrt(jnp.sum(q * q, axis=1)), 1e-12)[:, None]


def _score(qn, cbn):
    return lax.dot_general(qn.astype(jnp.bfloat16), cbn.astype(jnp.bfloat16),
                           (((1,), (1,)), ((), ())),
                           preferred_element_type=jnp.float32)


def _argmax0_kernel(q_ref, cbn_ref, idx_ref):
    qn = _normalize_rows(q_ref[...])
    s = _score(qn, cbn_ref[...])
    idx_ref[0, 0, :] = jnp.argmax(s, axis=1).astype(jnp.int32)


def _argmax0(ds0, cbn):
    return pl.pallas_call(
        _argmax0_kernel,
        out_specs=pl.BlockSpec((1, 1, _B), lambda: (0, 0, 0)),
        out_shape=jax.ShapeDtypeStruct((1, 1, _B), jnp.int32),
    )(ds0, cbn)


def _sc_gather(codebook, idx):
    info = plsc.get_sparse_core_info()
    nw = info.num_cores * info.num_subcores
    n = idx.shape[0]
    nw_used = min(nw, n // 8)
    b_per_w = n // nw_used
    mesh = plsc.VectorSubcoreMesh(core_axis_name="c", subcore_axis_name="s")

    @functools.partial(
        pl.kernel, mesh=mesh,
        out_type=jax.ShapeDtypeStruct((n, _C), jnp.float32),
        scratch_types=[
            pltpu.VMEM((b_per_w,), jnp.int32),
            pltpu.VMEM((b_per_w, _C), jnp.float32),
            pltpu.SemaphoreType.DMA,
        ],
    )
    def k(table_hbm, idx_hbm, out_hbm, idx_v, rows_v, sem):
        wid = lax.axis_index("s") * info.num_cores + lax.axis_index("c")

        @pl.when(wid < nw_used)
        def _():
            base = wid * b_per_w
            pltpu.sync_copy(idx_hbm.at[pl.ds(base, b_per_w)], idx_v)
            pltpu.async_copy(table_hbm.at[idx_v], rows_v, sem).wait()
            pltpu.sync_copy(rows_v, out_hbm.at[pl.ds(base, b_per_w)])

    return k(codebook, idx)


def _shift_clamp(x, off):
    n = x.shape[0]
    if off == 0:
        return x
    if off > 0:
        o = min(off, n)
        edge = jnp.broadcast_to(x[0:1, :], (o, x.shape[1]))
        if o == n:
            return edge
        return jnp.concatenate([edge, x[: n - o, :]], axis=0)
    o = min(-off, n)
    edge = jnp.broadcast_to(x[n - 1 : n, :], (o, x.shape[1]))
    if o == n:
        return edge
    return jnp.concatenate([x[o:, :], edge], axis=0)


def _upsample_vpu(pn, g, uw_ref):
    r = _H // pn
    g_exp = jnp.broadcast_to(g[:, None, :], (pn, r, _C)).reshape(_H, _C)
    gp = None
    for t in range(4):
        off = r // 2 - (t - 1) * r
        term = uw_ref[:, t][:, None] * _shift_clamp(g_exp, off)
        gp = term if gp is None else gp + term
    return gp


def _phi_update(si, g, uw_ref, w_ref, b_ref, rest_blk):
    if si != _NSC - 1:
        gp = _upsample_vpu(_PNS[si], g, uw_ref)
    else:
        gp = g
    gpb = gp.astype(jnp.bfloat16)
    zrow = jnp.zeros((1, _C), jnp.bfloat16)
    sd = jnp.concatenate([zrow, gpb[:-1, :]], axis=0)
    su = jnp.concatenate([gpb[1:, :], zrow], axis=0)
    mm = lambda x, w: lax.dot_general(
        x, w, (((1,), (0,)), ((), ())), preferred_element_type=jnp.float32)
    w = w_ref[...].astype(jnp.bfloat16)
    y2 = mm(sd, w[0]) + mm(gpb, w[1]) + mm(su, w[2])
    h = 0.5 * gp + 0.5 * (y2 + b_ref[0, :][None, :])
    return rest_blk - h


def _merged_kernel(si, pn, pn_next, nkb, g_ref, uw_ref, w_ref, b_ref,
                   rest_ref, cbn_ref, rest_out, idx_out, ss_out,
                   ds_s, m_s, a_s):
    b = pl.program_id(0)
    k = pl.program_id(1)
    cbs = _VOCAB // nkb

    @pl.when(k == 0)
    def _():
        rnew = _phi_update(si, g_ref[0], uw_ref, w_ref, b_ref, rest_ref[0])
        rest_out[0] = rnew
        r_next = _H // pn_next
        if r_next > 1:
            dsv = jnp.mean(rnew.reshape(pn_next, r_next, _C), axis=1)
        else:
            dsv = rnew
        ds_s[...] = _normalize_rows(dsv)
        ssq = jnp.sum(rnew * rnew)

        @pl.when(b == 0)
        def _():
            ss_out[0, 0] = ssq

        @pl.when(b > 0)
        def _():
            ss_out[0, 0] += ssq

    s = _score(ds_s[...], cbn_ref[pl.ds(k * cbs, cbs), :])
    lmax = jnp.max(s, axis=1)
    larg = jnp.argmax(s, axis=1).astype(jnp.int32) + k * cbs

    @pl.when(k == 0)
    def _():
        m_s[0, :] = lmax
        a_s[0, :] = larg

    @pl.when(k > 0)
    def _():
        m = m_s[0, :]
        better = lmax > m
        m_s[0, :] = jnp.where(better, lmax, m)
        a_s[0, :] = jnp.where(better, larg, a_s[0, :])

    @pl.when(k == nkb - 1)
    def _():
        idx_ref = a_s[0, :]
        idx_out[0, 0, :] = idx_ref


def _merged(si, g, rest, cbn, uw, w3, bias):
    pn = _PNS[si]
    pn_next = _PNS[si + 1]
    cbs = _code_tile(pn_next)
    nkb = _VOCAB // cbs

    body = functools.partial(_merged_kernel, si, pn, pn_next, nkb)
    return pl.pallas_call(
        body,
        grid=(_B, nkb),
        in_specs=[
            pl.BlockSpec((1, pn, _C), lambda b, k: (b, 0, 0)),
            pl.BlockSpec((_H, 4), lambda b, k: (0, 0)),
            pl.BlockSpec((3, _C, _C), lambda b, k: (0, 0, 0)),
            pl.BlockSpec((1, _C), lambda b, k: (0, 0)),
            pl.BlockSpec((1, _H, _C), lambda b, k: (b, 0, 0)),
            pl.BlockSpec((_VOCAB, _C), lambda b, k: (0, 0)),
        ],
        out_specs=[
            pl.BlockSpec((1, _H, _C), lambda b, k: (b, 0, 0)),
            pl.BlockSpec((1, 1, pn_next), lambda b, k: (b, 0, 0)),
            pl.BlockSpec((1, 1), lambda b, k: (0, 0), memory_space=pltpu.SMEM),
        ],
        out_shape=[
            jax.ShapeDtypeStruct((_B, _H, _C), jnp.float32),
            jax.ShapeDtypeStruct((_B, 1, pn_next), jnp.int32),
            jax.ShapeDtypeStruct((1, 1), jnp.float32),
        ],
        scratch_shapes=[
            pltpu.VMEM((pn_next, _C), jnp.float32),
            pltpu.VMEM((1, pn_next), jnp.float32),
            pltpu.VMEM((1, pn_next), jnp.int32),
        ],
    )(g.reshape(_B, pn, _C), uw, w3, bias, rest, cbn)


def _last_kernel(g_ref, w_ref, b_ref, rest_ref, f_ref, ss_out, fhat_out):
    b = pl.program_id(0)
    rnew = _phi_update(_NSC - 1, g_ref[0], None, w_ref, b_ref, rest_ref[0])
    fhat_out[0] = f_ref[0] - jnp.transpose(rnew)
    ssq = jnp.sum(rnew * rnew)

    @pl.when(b == 0)
    def _():
        ss_out[0, 0] = ssq

    @pl.when(b > 0)
    def _():
        ss_out[0, 0] += ssq


def _update_last(g, rest, f_nat, w3, bias):
    pn = _PNS[-1]
    return pl.pallas_call(
        _last_kernel,
        grid=(_B,),
        in_specs=[
            pl.BlockSpec((1, pn, _C), lambda b: (b, 0, 0)),
            pl.BlockSpec((3, _C, _C), lambda b: (0, 0, 0)),
            pl.BlockSpec((1, _C), lambda b: (0, 0)),
            pl.BlockSpec((1, _H, _C), lambda b: (b, 0, 0)),
            pl.BlockSpec((1, _C, _H), lambda b: (b, 0, 0)),
        ],
        out_specs=[
            pl.BlockSpec((1, 1), lambda b: (0, 0), memory_space=pltpu.SMEM),
            pl.BlockSpec((1, _C, _H), lambda b: (b, 0, 0)),
        ],
        out_shape=[
            jax.ShapeDtypeStruct((1, 1), jnp.float32),
            jax.ShapeDtypeStruct((_B, _C, _H), jnp.float32),
        ],
    )(g.reshape(_B, pn, _C), w3, bias, rest, f_nat)


def kernel(f_BChw, codebook, phi_w, phi_b):
    f_nat = f_BChw.reshape(_B, _C, _H)

    cbn, ds0, f_r = _prologue(codebook, f_nat)

    w3s, biases = [], []
    for si in range(_NSC):
        k = _phi_share(si)
        w3s.append(jnp.transpose(phi_w[k, :, :, :, 1], (2, 1, 0)))
        biases.append(phi_b[k].reshape(1, _C))

    rest = f_r
    idx = _argmax0(ds0.reshape(_B, _C), cbn).reshape(_B)
    ss_list = []
    for si in range(_NSC - 1):
        g = _sc_gather(codebook, idx)
        uw = jnp.asarray(_upsample_weights(_PNS[si], _H))
        rest, idx3, ss = _merged(si, g, rest, cbn, uw, w3s[si], biases[si])
        idx = idx3.reshape(_B * _PNS[si + 1])
        ss_list.append(ss[0, 0])

    g = _sc_gather(codebook, idx)
    ss9, fhat = _update_last(g, rest, f_nat, w3s[-1], biases[-1])
    ss_list.append(ss9[0, 0])

    numel = _B * _H * _C
    loss = (1.0 + _BETA) / _NSC * jnp.sum(jnp.stack(ss_list)) / numel
    f_hat_out = fhat.reshape(_B, _C, _H, 1)
    return (f_hat_out, loss)


# speedup vs baseline: 1.7134x; 1.2095x over previous
import functools

import jax
import jax.numpy as jnp
import numpy as np
from jax import lax
from jax.experimental import pallas as pl
from jax.experimental.pallas import tpu as pltpu
from jax.experimental.pallas import tpu_sc as plsc

_PNS = (1, 2, 4, 8, 16, 32, 64, 128, 256, 512)
_VOCAB = 8192
_C = 256
_B = 16
_H = 512
_BETA = 0.25
_SHARE = 4
_NSC = 10


def _cubic_w_np(t, a=-0.75):
    at = np.abs(t)
    w1 = (a + 2.0) * at ** 3 - (a + 3.0) * at ** 2 + 1.0
    w2 = a * at ** 3 - 5.0 * a * at ** 2 + 8.0 * a * at - 4.0 * a
    return np.where(at <= 1.0, w1, np.where(at < 2.0, w2, np.zeros_like(at)))


@functools.lru_cache(maxsize=None)
def _upsample_weights(pn: int, out_h: int) -> np.ndarray:
    scale = pn / out_h
    i = np.arange(out_h, dtype=np.float32)
    src = (i + 0.5) * scale - 0.5
    i0 = np.floor(src).astype(np.int32)
    w = np.zeros((out_h, 4), dtype=np.float32)
    for t in range(4):
        tap = i0 - 1 + t
        w[:, t] = _cubic_w_np((src - tap).astype(np.float32))
    return w


@functools.lru_cache(maxsize=None)
def _phi_share(si: int) -> int:
    ticks = np.linspace(1.0 / 3.0 / _SHARE, 1.0 - 1.0 / 3.0 / _SHARE, _SHARE)
    return int(np.argmin(np.abs(ticks - si / (_NSC - 1))))


def _code_tile(pn_next: int) -> int:
    return _VOCAB if pn_next <= 64 else 2048


def _prologue_kernel(cb_ref, f_ref, cbn_ref, ds_ref, fr_ref):
    cb = cb_ref[...]
    nrm = jnp.sqrt(jnp.sum(cb * cb, axis=1))
    cbn_ref[...] = cb / jnp.maximum(nrm, 1e-12)[:, None]
    fb = f_ref[0]
    ds_ref[0, 0, :] = jnp.mean(fb, axis=1)
    fr_ref[0] = jnp.transpose(fb)


def _prologue(codebook, f_nat):
    return pl.pallas_call(
        _prologue_kernel,
        grid=(_B,),
        in_specs=[
            pl.BlockSpec((_VOCAB // _B, _C), lambda i: (i, 0)),
            pl.BlockSpec((1, _C, _H), lambda i: (i, 0, 0)),
        ],
        out_specs=[
            pl.BlockSpec((_VOCAB // _B, _C), lambda i: (i, 0)),
            pl.BlockSpec((1, 1, _C), lambda i: (i, 0, 0)),
            pl.BlockSpec((1, _H, _C), lambda i: (i, 0, 0)),
        ],
        out_shape=[
            jax.ShapeDtypeStruct((_VOCAB, _C), jnp.float32),
            jax.ShapeDtypeStruct((_B, 1, _C), jnp.float32),
            jax.ShapeDtypeStruct((_B, _H, _C), jnp.float32),
        ],
    )(codebook, f_nat)


def _normalize_rows(q):
    return q / jnp.maximum(jnp.sqrt(jnp.sum(q * q, axis=1)), 1e-12)[:, None]


def _score(qn, cbn):
    return lax.dot_general(qn.astype(jnp.bfloat16), cbn.astype(jnp.bfloat16),
                           (((1,), (1,)), ((), ())),
                           preferred_element_type=jnp.float32)


def _argmax0_kernel(q_ref, cbn_ref, idx_ref):
    qn = _normalize_rows(q_ref[...])
    s = _score(qn, cbn_ref[...])
    idx_ref[0, 0, :] = jnp.argmax(s, axis=1).astype(jnp.int32)


def _argmax0(ds0, cbn):
    return pl.pallas_call(
        _argmax0_kernel,
        out_specs=pl.BlockSpec((1, 1, _B), lambda: (0, 0, 0)),
        out_shape=jax.ShapeDtypeStruct((1, 1, _B), jnp.int32),
    )(ds0, cbn)


def _sc_gather(codebook, idx):
    info = plsc.get_sparse_core_info()
    nw = info.num_cores * info.num_subcores
    n = idx.shape[0]
    nw_used = min(nw, n // 8)
    b_per_w = n // nw_used
    mesh = plsc.VectorSubcoreMesh(core_axis_name="c", subcore_axis_name="s")

    @functools.partial(
        pl.kernel, mesh=mesh,
        out_type=jax.ShapeDtypeStruct((n, _C), jnp.float32),
        scratch_types=[
            pltpu.VMEM((b_per_w,), jnp.int32),
            pltpu.VMEM((b_per_w, _C), jnp.float32),
            pltpu.SemaphoreType.DMA,
        ],
    )
    def k(table_hbm, idx_hbm, out_hbm, idx_v, rows_v, sem):
        wid = lax.axis_index("s") * info.num_cores + lax.axis_index("c")

        @pl.when(wid < nw_used)
        def _():
            base = wid * b_per_w
            pltpu.sync_copy(idx_hbm.at[pl.ds(base, b_per_w)], idx_v)
            pltpu.async_copy(table_hbm.at[idx_v], rows_v, sem).wait()
            pltpu.sync_copy(rows_v, out_hbm.at[pl.ds(base, b_per_w)])

    return k(codebook, idx)


def _shift_clamp(x, off):
    n = x.shape[0]
    if off == 0:
        return x
    if off > 0:
        o = min(off, n)
        edge = jnp.broadcast_to(x[0:1, :], (o, x.shape[1]))
        if o == n:
            return edge
        return jnp.concatenate([edge, x[: n - o, :]], axis=0)
    o = min(-off, n)
    edge = jnp.broadcast_to(x[n - 1 : n, :], (o, x.shape[1]))
    if o == n:
        return edge
    return jnp.concatenate([x[o:, :], edge], axis=0)


def _upsample_vpu(pn, g, uw_ref):
    r = _H // pn
    g_exp = jnp.broadcast_to(g[:, None, :], (pn, r, _C)).reshape(_H, _C)
    gp = None
    for t in range(4):
        off = r // 2 - (t - 1) * r
        term = uw_ref[:, t][:, None] * _shift_clamp(g_exp, off)
        gp = term if gp is None else gp + term
    return gp


def _phi_update(si, g, uw_ref, w_ref, b_ref, rest_blk):
    if si != _NSC - 1:
        gp = _upsample_vpu(_PNS[si], g, uw_ref)
    else:
        gp = g
    gpb = gp.astype(jnp.bfloat16)
    zrow = jnp.zeros((1, _C), jnp.bfloat16)
    sd = jnp.concatenate([zrow, gpb[:-1, :]], axis=0)
    su = jnp.concatenate([gpb[1:, :], zrow], axis=0)
    mm = lambda x, w: lax.dot_general(
        x, w, (((1,), (0,)), ((), ())), preferred_element_type=jnp.float32)
    w = w_ref[...].astype(jnp.bfloat16)
    y2 = mm(sd, w[0]) + mm(gpb, w[1]) + mm(su, w[2])
    h = 0.5 * gp + 0.5 * (y2 + b_ref[0, :][None, :])
    return rest_blk - h


def _merged_kernel(si, pn, pn_next, nkb, g_ref, uw_ref, w_ref, b_ref,
                   rest_ref, cbn_ref, rest_out, idx_out, ss_out,
                   ds_s, m_s, a_s):
    b = pl.program_id(0)
    k = pl.program_id(1)
    cbs = _VOCAB // nkb

    @pl.when(k == 0)
    def _():
        rnew = _phi_update(si, g_ref[0], uw_ref, w_ref, b_ref, rest_ref[0])
        rest_out[0] = rnew
        r_next = _H // pn_next
        if r_next > 1:
            dsv = jnp.mean(rnew.reshape(pn_next, r_next, _C), axis=1)
        else:
            dsv = rnew
        ds_s[...] = _normalize_rows(dsv)
        ssq = jnp.sum(rnew * rnew)

        @pl.when(b == 0)
        def _():
            ss_out[0, 0] = ssq

        @pl.when(b > 0)
        def _():
            ss_out[0, 0] += ssq

    s = _score(ds_s[...], cbn_ref[pl.ds(k * cbs, cbs), :])
    lmax = jnp.max(s, axis=1)
    larg = jnp.argmax(s, axis=1).astype(jnp.int32) + k * cbs

    @pl.when(k == 0)
    def _():
        m_s[0, :] = lmax
        a_s[0, :] = larg

    @pl.when(k > 0)
    def _():
        m = m_s[0, :]
        better = lmax > m
        m_s[0, :] = jnp.where(better, lmax, m)
        a_s[0, :] = jnp.where(better, larg, a_s[0, :])

    @pl.when(k == nkb - 1)
    def _():
        idx_ref = a_s[0, :]
        idx_out[0, 0, :] = idx_ref


def _merged_small_kernel(si, pn, pn_next, g_ref, uw_ref, w_ref, b_ref,
                         rest_ref, cbn_ref, rest_out, idx_out, ss_out):
    r_next = _H // pn_next
    ss = None
    ds_list = []
    for b in range(_B):
        rnew = _phi_update(si, g_ref[b], uw_ref, w_ref, b_ref, rest_ref[b])
        rest_out[b] = rnew
        ds_list.append(jnp.mean(rnew.reshape(pn_next, r_next, _C), axis=1))
        ssq = jnp.sum(rnew * rnew)
        ss = ssq if ss is None else ss + ssq
    ss_out[0, 0] = ss
    qn = _normalize_rows(jnp.concatenate(ds_list, axis=0))
    cbs = 2048
    m = None
    a = None
    for k in range(_VOCAB // cbs):
        s = _score(qn, cbn_ref[k * cbs:(k + 1) * cbs, :])
        lmax = jnp.max(s, axis=1)
        larg = jnp.argmax(s, axis=1).astype(jnp.int32) + k * cbs
        if m is None:
            m, a = lmax, larg
        else:
            better = lmax > m
            m = jnp.where(better, lmax, m)
            a = jnp.where(better, larg, a)
    idx_out[0, 0, :] = a


def _merged_small(si, g, rest, cbn, uw, w3, bias):
    pn = _PNS[si]
    pn_next = _PNS[si + 1]
    n_next = _B * pn_next

    body = functools.partial(_merged_small_kernel, si, pn, pn_next)
    return pl.pallas_call(
        body,
        in_specs=[
            pl.BlockSpec((_B, pn, _C), lambda: (0, 0, 0)),
            pl.BlockSpec((_H, 4), lambda: (0, 0)),
            pl.BlockSpec((3, _C, _C), lambda: (0, 0, 0)),
            pl.BlockSpec((1, _C), lambda: (0, 0)),
            pl.BlockSpec((_B, _H, _C), lambda: (0, 0, 0)),
            pl.BlockSpec((_VOCAB, _C), lambda: (0, 0)),
        ],
        out_specs=[
            pl.BlockSpec((_B, _H, _C), lambda: (0, 0, 0)),
            pl.BlockSpec((1, 1, n_next), lambda: (0, 0, 0)),
            pl.BlockSpec((1, 1), lambda: (0, 0), memory_space=pltpu.SMEM),
        ],
        out_shape=[
            jax.ShapeDtypeStruct((_B, _H, _C), jnp.float32),
            jax.ShapeDtypeStruct((1, 1, n_next), jnp.int32),
            jax.ShapeDtypeStruct((1, 1), jnp.float32),
        ],
    )(g.reshape(_B, pn, _C), uw, w3, bias, rest, cbn)


def _merged(si, g, rest, cbn, uw, w3, bias):
    pn = _PNS[si]
    pn_next = _PNS[si + 1]
    cbs = _code_tile(pn_next)
    nkb = _VOCAB // cbs

    body = functools.partial(_merged_kernel, si, pn, pn_next, nkb)
    return pl.pallas_call(
        body,
        grid=(_B, nkb),
        in_specs=[
            pl.BlockSpec((1, pn, _C), lambda b, k: (b, 0, 0)),
            pl.BlockSpec((_H, 4), lambda b, k: (0, 0)),
            pl.BlockSpec((3, _C, _C), lambda b, k: (0, 0, 0)),
            pl.BlockSpec((1, _C), lambda b, k: (0, 0)),
            pl.BlockSpec((1, _H, _C), lambda b, k: (b, 0, 0)),
            pl.BlockSpec((_VOCAB, _C), lambda b, k: (0, 0)),
        ],
        out_specs=[
            pl.BlockSpec((1, _H, _C), lambda b, k: (b, 0, 0)),
            pl.BlockSpec((1, 1, pn_next), lambda b, k: (b, 0, 0)),
            pl.BlockSpec((1, 1), lambda b, k: (0, 0), memory_space=pltpu.SMEM),
        ],
        out_shape=[
            jax.ShapeDtypeStruct((_B, _H, _C), jnp.float32),
            jax.ShapeDtypeStruct((_B, 1, pn_next), jnp.int32),
            jax.ShapeDtypeStruct((1, 1), jnp.float32),
        ],
        scratch_shapes=[
            pltpu.VMEM((pn_next, _C), jnp.float32),
            pltpu.VMEM((1, pn_next), jnp.float32),
            pltpu.VMEM((1, pn_next), jnp.int32),
        ],
    )(g.reshape(_B, pn, _C), uw, w3, bias, rest, cbn)


def _last_kernel(g_ref, w_ref, b_ref, rest_ref, f_ref, ss_out, fhat_out):
    b = pl.program_id(0)
    rnew = _phi_update(_NSC - 1, g_ref[0], None, w_ref, b_ref, rest_ref[0])
    fhat_out[0] = f_ref[0] - jnp.transpose(rnew)
    ssq = jnp.sum(rnew * rnew)

    @pl.when(b == 0)
    def _():
        ss_out[0, 0] = ssq

    @pl.when(b > 0)
    def _():
        ss_out[0, 0] += ssq


def _update_last(g, rest, f_nat, w3, bias):
    pn = _PNS[-1]
    return pl.pallas_call(
        _last_kernel,
        grid=(_B,),
        in_specs=[
            pl.BlockSpec((1, pn, _C), lambda b: (b, 0, 0)),
            pl.BlockSpec((3, _C, _C), lambda b: (0, 0, 0)),
            pl.BlockSpec((1, _C), lambda b: (0, 0)),
            pl.BlockSpec((1, _H, _C), lambda b: (b, 0, 0)),
            pl.BlockSpec((1, _C, _H), lambda b: (b, 0, 0)),
        ],
        out_specs=[
            pl.BlockSpec((1, 1), lambda b: (0, 0), memory_space=pltpu.SMEM),
            pl.BlockSpec((1, _C, _H), lambda b: (b, 0, 0)),
        ],
        out_shape=[
            jax.ShapeDtypeStruct((1, 1), jnp.float32),
            jax.ShapeDtypeStruct((_B, _C, _H), jnp.float32),
        ],
    )(g.reshape(_B, pn, _C), w3, bias, rest, f_nat)


def kernel(f_BChw, codebook, phi_w, phi_b):
    f_nat = f_BChw.reshape(_B, _C, _H)

    cbn, ds0, f_r = _prologue(codebook, f_nat)

    w3s, biases = [], []
    for si in range(_NSC):
        k = _phi_share(si)
        w3s.append(jnp.transpose(phi_w[k, :, :, :, 1], (2, 1, 0)))
        biases.append(phi_b[k].reshape(1, _C))

    rest = f_r
    idx = _argmax0(ds0.reshape(_B, _C), cbn).reshape(_B)
    ss_list = []
    for si in range(_NSC - 1):
        g = _sc_gather(codebook, idx)
        uw = jnp.asarray(_upsample_weights(_PNS[si], _H))
        fn = _merged_small if _PNS[si + 1] <= 64 else _merged
        rest, idx3, ss = fn(si, g, rest, cbn, uw, w3s[si], biases[si])
        idx = idx3.reshape(_B * _PNS[si + 1])
        ss_list.append(ss[0, 0])

    g = _sc_gather(codebook, idx)
    ss9, fhat = _update_last(g, rest, f_nat, w3s[-1], biases[-1])
    ss_list.append(ss9[0, 0])

    numel = _B * _H * _C
    loss = (1.0 + _BETA) / _NSC * jnp.sum(jnp.stack(ss_list)) / numel
    f_hat_out = fhat.reshape(_B, _C, _H, 1)
    return (f_hat_out, loss)


# speedup vs baseline: 2.2498x; 1.3131x over previous
import functools

import jax
import jax.numpy as jnp
import numpy as np
from jax import lax
from jax.experimental import pallas as pl
from jax.experimental.pallas import tpu as pltpu
from jax.experimental.pallas import tpu_sc as plsc

_PNS = (1, 2, 4, 8, 16, 32, 64, 128, 256, 512)
_VOCAB = 8192
_C = 256
_B = 16
_H = 512
_BETA = 0.25
_SHARE = 4
_NSC = 10


def _cubic_w_np(t, a=-0.75):
    at = np.abs(t)
    w1 = (a + 2.0) * at ** 3 - (a + 3.0) * at ** 2 + 1.0
    w2 = a * at ** 3 - 5.0 * a * at ** 2 + 8.0 * a * at - 4.0 * a
    return np.where(at <= 1.0, w1, np.where(at < 2.0, w2, np.zeros_like(at)))


@functools.lru_cache(maxsize=None)
def _upsample_weights(pn: int, out_h: int) -> np.ndarray:
    scale = pn / out_h
    i = np.arange(out_h, dtype=np.float32)
    src = (i + 0.5) * scale - 0.5
    i0 = np.floor(src).astype(np.int32)
    w = np.zeros((out_h, 4), dtype=np.float32)
    for t in range(4):
        tap = i0 - 1 + t
        w[:, t] = _cubic_w_np((src - tap).astype(np.float32))
    return w


@functools.lru_cache(maxsize=None)
def _phi_share(si: int) -> int:
    ticks = np.linspace(1.0 / 3.0 / _SHARE, 1.0 - 1.0 / 3.0 / _SHARE, _SHARE)
    return int(np.argmin(np.abs(ticks - si / (_NSC - 1))))


def _code_tile(pn_next: int) -> int:
    return _VOCAB if pn_next <= 64 else 2048


def _prologue_kernel(cb_ref, f_ref, cbn_ref, ds_ref, fr_ref):
    cb = cb_ref[...]
    nrm = jnp.sqrt(jnp.sum(cb * cb, axis=1))
    cbn_ref[...] = cb / jnp.maximum(nrm, 1e-12)[:, None]
    fb = f_ref[0]
    ds_ref[0, 0, :] = jnp.mean(fb, axis=1)
    fr_ref[0] = jnp.transpose(fb)


def _prologue(codebook, f_nat):
    return pl.pallas_call(
        _prologue_kernel,
        grid=(_B,),
        in_specs=[
            pl.BlockSpec((_VOCAB // _B, _C), lambda i: (i, 0)),
            pl.BlockSpec((1, _C, _H), lambda i: (i, 0, 0)),
        ],
        out_specs=[
            pl.BlockSpec((_VOCAB // _B, _C), lambda i: (i, 0)),
            pl.BlockSpec((1, 1, _C), lambda i: (i, 0, 0)),
            pl.BlockSpec((1, _H, _C), lambda i: (i, 0, 0)),
        ],
        out_shape=[
            jax.ShapeDtypeStruct((_VOCAB, _C), jnp.float32),
            jax.ShapeDtypeStruct((_B, 1, _C), jnp.float32),
            jax.ShapeDtypeStruct((_B, _H, _C), jnp.float32),
        ],
    )(codebook, f_nat)


def _normalize_rows(q):
    return q / jnp.maximum(jnp.sqrt(jnp.sum(q * q, axis=1)), 1e-12)[:, None]


def _score(qn, cbn):
    return lax.dot_general(qn.astype(jnp.bfloat16), cbn.astype(jnp.bfloat16),
                           (((1,), (1,)), ((), ())),
                           preferred_element_type=jnp.float32)


def _argmax0_kernel(q_ref, cbn_ref, idx_ref):
    qn = _normalize_rows(q_ref[...])
    s = _score(qn, cbn_ref[...])
    idx_ref[0, 0, :] = jnp.argmax(s, axis=1).astype(jnp.int32)


def _argmax0(ds0, cbn):
    return pl.pallas_call(
        _argmax0_kernel,
        out_specs=pl.BlockSpec((1, 1, _B), lambda: (0, 0, 0)),
        out_shape=jax.ShapeDtypeStruct((1, 1, _B), jnp.int32),
    )(ds0, cbn)


def _sc_gather(codebook, idx):
    info = plsc.get_sparse_core_info()
    nw = info.num_cores * info.num_subcores
    n = idx.shape[0]
    nw_used = min(nw, n // 8)
    b_per_w = n // nw_used
    mesh = plsc.VectorSubcoreMesh(core_axis_name="c", subcore_axis_name="s")

    @functools.partial(
        pl.kernel, mesh=mesh,
        out_type=jax.ShapeDtypeStruct((n, _C), jnp.float32),
        scratch_types=[
            pltpu.VMEM((b_per_w,), jnp.int32),
            pltpu.VMEM((b_per_w, _C), jnp.float32),
            pltpu.SemaphoreType.DMA,
        ],
    )
    def k(table_hbm, idx_hbm, out_hbm, idx_v, rows_v, sem):
        wid = lax.axis_index("s") * info.num_cores + lax.axis_index("c")

        @pl.when(wid < nw_used)
        def _():
            base = wid * b_per_w
            pltpu.sync_copy(idx_hbm.at[pl.ds(base, b_per_w)], idx_v)
            pltpu.async_copy(table_hbm.at[idx_v], rows_v, sem).wait()
            pltpu.sync_copy(rows_v, out_hbm.at[pl.ds(base, b_per_w)])

    return k(codebook, idx)


def _shift_clamp(x, off):
    n = x.shape[0]
    if off == 0:
        return x
    if off > 0:
        o = min(off, n)
        edge = jnp.broadcast_to(x[0:1, :], (o, x.shape[1]))
        if o == n:
            return edge
        return jnp.concatenate([edge, x[: n - o, :]], axis=0)
    o = min(-off, n)
    edge = jnp.broadcast_to(x[n - 1 : n, :], (o, x.shape[1]))
    if o == n:
        return edge
    return jnp.concatenate([x[o:, :], edge], axis=0)


def _upsample_vpu(pn, g, uw_ref):
    r = _H // pn
    g_exp = jnp.broadcast_to(g[:, None, :], (pn, r, _C)).reshape(_H, _C)
    gp = None
    for t in range(4):
        off = r // 2 - (t - 1) * r
        term = uw_ref[:, t][:, None] * _shift_clamp(g_exp, off)
        gp = term if gp is None else gp + term
    return gp


def _phi_update(si, g, uw_ref, w_ref, b_ref, rest_blk):
    if si != _NSC - 1:
        gp = _upsample_vpu(_PNS[si], g, uw_ref)
    else:
        gp = g
    gpb = gp.astype(jnp.bfloat16)
    zrow = jnp.zeros((1, _C), jnp.bfloat16)
    sd = jnp.concatenate([zrow, gpb[:-1, :]], axis=0)
    su = jnp.concatenate([gpb[1:, :], zrow], axis=0)
    mm = lambda x, w: lax.dot_general(
        x, w, (((1,), (0,)), ((), ())), preferred_element_type=jnp.float32)
    w = w_ref[...].astype(jnp.bfloat16)
    y2 = mm(sd, w[0]) + mm(gpb, w[1]) + mm(su, w[2])
    h = 0.5 * gp + 0.5 * (y2 + b_ref[0, :][None, :])
    return rest_blk - h


def _merged_kernel(si, pn, pn_next, bb, nrb, g_ref, uw_ref, w_ref, b_ref,
                   rest_ref, cbn_ref, rest_out, idx_out, ss_out):
    i = pl.program_id(0) if nrb > 1 else 0
    r_next = _H // pn_next
    ss = None
    ds_list = []
    for b in range(bb):
        rnew = _phi_update(si, g_ref[b], uw_ref, w_ref, b_ref, rest_ref[b])
        rest_out[b] = rnew
        if r_next > 1:
            ds_list.append(jnp.mean(rnew.reshape(pn_next, r_next, _C), axis=1))
        else:
            ds_list.append(rnew)
        ssq = jnp.sum(rnew * rnew)
        ss = ssq if ss is None else ss + ssq

    if nrb == 1:
        ss_out[0, 0] = ss
    else:
        @pl.when(i == 0)
        def _():
            ss_out[0, 0] = ss

        @pl.when(i > 0)
        def _():
            ss_out[0, 0] += ss

    qn = _normalize_rows(jnp.concatenate(ds_list, axis=0)
                         if bb > 1 else ds_list[0])
    cbs = 2048
    m = None
    a = None
    for k in range(_VOCAB // cbs):
        s = _score(qn, cbn_ref[k * cbs:(k + 1) * cbs, :])
        lmax = jnp.max(s, axis=1)
        larg = jnp.argmax(s, axis=1).astype(jnp.int32) + k * cbs
        if m is None:
            m, a = lmax, larg
        else:
            better = lmax > m
            m = jnp.where(better, lmax, m)
            a = jnp.where(better, larg, a)
    idx_out[0, 0, :] = a


def _merged(si, g, rest, cbn, uw, w3, bias):
    pn = _PNS[si]
    pn_next = _PNS[si + 1]
    bb = min(_B, max(1, 1024 // pn_next))
    nrb = _B // bb
    qrows = bb * pn_next

    body = functools.partial(_merged_kernel, si, pn, pn_next, bb, nrb)
    return pl.pallas_call(
        body,
        grid=(nrb,),
        in_specs=[
            pl.BlockSpec((bb, pn, _C), lambda i: (i, 0, 0)),
            pl.BlockSpec((_H, 4), lambda i: (0, 0)),
            pl.BlockSpec((3, _C, _C), lambda i: (0, 0, 0)),
            pl.BlockSpec((1, _C), lambda i: (0, 0)),
            pl.BlockSpec((bb, _H, _C), lambda i: (i, 0, 0)),
            pl.BlockSpec((_VOCAB, _C), lambda i: (0, 0)),
        ],
        out_specs=[
            pl.BlockSpec((bb, _H, _C), lambda i: (i, 0, 0)),
            pl.BlockSpec((1, 1, qrows), lambda i: (i, 0, 0)),
            pl.BlockSpec((1, 1), lambda i: (0, 0), memory_space=pltpu.SMEM),
        ],
        out_shape=[
            jax.ShapeDtypeStruct((_B, _H, _C), jnp.float32),
            jax.ShapeDtypeStruct((nrb, 1, qrows), jnp.int32),
            jax.ShapeDtypeStruct((1, 1), jnp.float32),
        ],
    )(g.reshape(_B, pn, _C), uw, w3, bias, rest, cbn)


def _last_kernel(g_ref, w_ref, b_ref, rest_ref, f_ref, ss_out, fhat_out):
    ss = None
    for b in range(_B):
        rnew = _phi_update(_NSC - 1, g_ref[b], None, w_ref, b_ref, rest_ref[b])
        fhat_out[b] = f_ref[b] - jnp.transpose(rnew)
        ssq = jnp.sum(rnew * rnew)
        ss = ssq if ss is None else ss + ssq
    ss_out[0, 0] = ss


def _update_last(g, rest, f_nat, w3, bias):
    pn = _PNS[-1]
    return pl.pallas_call(
        _last_kernel,
        in_specs=[
            pl.BlockSpec((_B, pn, _C), lambda: (0, 0, 0)),
            pl.BlockSpec((3, _C, _C), lambda: (0, 0, 0)),
            pl.BlockSpec((1, _C), lambda: (0, 0)),
            pl.BlockSpec((_B, _H, _C), lambda: (0, 0, 0)),
            pl.BlockSpec((_B, _C, _H), lambda: (0, 0, 0)),
        ],
        out_specs=[
            pl.BlockSpec((1, 1), lambda: (0, 0), memory_space=pltpu.SMEM),
            pl.BlockSpec((_B, _C, _H), lambda: (0, 0, 0)),
        ],
        out_shape=[
            jax.ShapeDtypeStruct((1, 1), jnp.float32),
            jax.ShapeDtypeStruct((_B, _C, _H), jnp.float32),
        ],
    )(g.reshape(_B, pn, _C), w3, bias, rest, f_nat)


def kernel(f_BChw, codebook, phi_w, phi_b):
    f_nat = f_BChw.reshape(_B, _C, _H)

    cbn, ds0, f_r = _prologue(codebook, f_nat)

    w3s, biases = [], []
    for si in range(_NSC):
        k = _phi_share(si)
        w3s.append(jnp.transpose(phi_w[k, :, :, :, 1], (2, 1, 0)))
        biases.append(phi_b[k].reshape(1, _C))

    rest = f_r
    idx = _argmax0(ds0.reshape(_B, _C), cbn).reshape(_B)
    ss_list = []
    for si in range(_NSC - 1):
        g = _sc_gather(codebook, idx)
        uw = jnp.asarray(_upsample_weights(_PNS[si], _H))
        rest, idx3, ss = _merged(si, g, rest, cbn, uw, w3s[si], biases[si])
        idx = idx3.reshape(_B * _PNS[si + 1])
        ss_list.append(ss[0, 0])

    g = _sc_gather(codebook, idx)
    ss9, fhat = _update_last(g, rest, f_nat, w3s[-1], biases[-1])
    ss_list.append(ss9[0, 0])

    numel = _B * _H * _C
    loss = (1.0 + _BETA) / _NSC * jnp.sum(jnp.stack(ss_list)) / numel
    f_hat_out = fhat.reshape(_B, _C, _H, 1)
    return (f_hat_out, loss)


# speedup vs baseline: 2.2503x; 1.0002x over previous
import functools

import jax
import jax.numpy as jnp
import numpy as np
from jax import lax
from jax.experimental import pallas as pl
from jax.experimental.pallas import tpu as pltpu
from jax.experimental.pallas import tpu_sc as plsc

_PNS = (1, 2, 4, 8, 16, 32, 64, 128, 256, 512)
_VOCAB = 8192
_C = 256
_B = 16
_H = 512
_BETA = 0.25
_SHARE = 4
_NSC = 10


def _cubic_w_np(t, a=-0.75):
    at = np.abs(t)
    w1 = (a + 2.0) * at ** 3 - (a + 3.0) * at ** 2 + 1.0
    w2 = a * at ** 3 - 5.0 * a * at ** 2 + 8.0 * a * at - 4.0 * a
    return np.where(at <= 1.0, w1, np.where(at < 2.0, w2, np.zeros_like(at)))


@functools.lru_cache(maxsize=None)
def _upsample_weights(pn: int, out_h: int) -> np.ndarray:
    scale = pn / out_h
    i = np.arange(out_h, dtype=np.float32)
    src = (i + 0.5) * scale - 0.5
    i0 = np.floor(src).astype(np.int32)
    w = np.zeros((out_h, 4), dtype=np.float32)
    for t in range(4):
        tap = i0 - 1 + t
        w[:, t] = _cubic_w_np((src - tap).astype(np.float32))
    return w


@functools.lru_cache(maxsize=None)
def _phi_share(si: int) -> int:
    ticks = np.linspace(1.0 / 3.0 / _SHARE, 1.0 - 1.0 / 3.0 / _SHARE, _SHARE)
    return int(np.argmin(np.abs(ticks - si / (_NSC - 1))))


def _code_tile(pn_next: int) -> int:
    return _VOCAB if pn_next <= 64 else 2048


def _prologue_kernel(cb_ref, f_ref, cbn_ref, ds_ref, fr_ref):
    cb = cb_ref[...]
    nrm = jnp.sqrt(jnp.sum(cb * cb, axis=1))
    cbn_ref[...] = cb / jnp.maximum(nrm, 1e-12)[:, None]
    fb = f_ref[0]
    ds_ref[0, 0, :] = jnp.mean(fb, axis=1)
    fr_ref[0] = jnp.transpose(fb)


def _prologue(codebook, f_nat):
    return pl.pallas_call(
        _prologue_kernel,
        grid=(_B,),
        in_specs=[
            pl.BlockSpec((_VOCAB // _B, _C), lambda i: (i, 0)),
            pl.BlockSpec((1, _C, _H), lambda i: (i, 0, 0)),
        ],
        out_specs=[
            pl.BlockSpec((_VOCAB // _B, _C), lambda i: (i, 0)),
            pl.BlockSpec((1, 1, _C), lambda i: (i, 0, 0)),
            pl.BlockSpec((1, _H, _C), lambda i: (i, 0, 0)),
        ],
        out_shape=[
            jax.ShapeDtypeStruct((_VOCAB, _C), jnp.float32),
            jax.ShapeDtypeStruct((_B, 1, _C), jnp.float32),
            jax.ShapeDtypeStruct((_B, _H, _C), jnp.float32),
        ],
    )(codebook, f_nat)


def _normalize_rows(q):
    return q / jnp.maximum(jnp.sqrt(jnp.sum(q * q, axis=1)), 1e-12)[:, None]


def _score(qn, cbn):
    return lax.dot_general(qn.astype(jnp.bfloat16), cbn.astype(jnp.bfloat16),
                           (((1,), (1,)), ((), ())),
                           preferred_element_type=jnp.float32)


def _argmax0_kernel(q_ref, cbn_ref, idx_ref):
    qn = _normalize_rows(q_ref[...])
    s = _score(qn, cbn_ref[...])
    idx_ref[0, 0, :] = jnp.argmax(s, axis=1).astype(jnp.int32)


def _argmax0(ds0, cbn):
    return pl.pallas_call(
        _argmax0_kernel,
        out_specs=pl.BlockSpec((1, 1, _B), lambda: (0, 0, 0)),
        out_shape=jax.ShapeDtypeStruct((1, 1, _B), jnp.int32),
    )(ds0, cbn)


def _sc_gather(codebook, idx):
    info = plsc.get_sparse_core_info()
    nw = info.num_cores * info.num_subcores
    n = idx.shape[0]
    nw_used = min(nw, n // 8)
    b_per_w = n // nw_used
    mesh = plsc.VectorSubcoreMesh(core_axis_name="c", subcore_axis_name="s")

    @functools.partial(
        pl.kernel, mesh=mesh,
        out_type=jax.ShapeDtypeStruct((n, _C), jnp.float32),
        scratch_types=[
            pltpu.VMEM((b_per_w,), jnp.int32),
            pltpu.VMEM((b_per_w, _C), jnp.float32),
            pltpu.SemaphoreType.DMA,
        ],
        compiler_params=pltpu.CompilerParams(use_tc_tiling_on_sc=True),
    )
    def k(table_hbm, idx_hbm, out_hbm, idx_v, rows_v, sem):
        wid = lax.axis_index("s") * info.num_cores + lax.axis_index("c")

        @pl.when(wid < nw_used)
        def _():
            base = wid * b_per_w
            pltpu.sync_copy(idx_hbm.at[pl.ds(base, b_per_w)], idx_v)
            pltpu.async_copy(table_hbm.at[idx_v], rows_v, sem).wait()
            pltpu.sync_copy(rows_v, out_hbm.at[pl.ds(base, b_per_w)])

    return k(codebook, idx)


def _shift_clamp(x, off):
    n = x.shape[0]
    if off == 0:
        return x
    if off > 0:
        o = min(off, n)
        edge = jnp.broadcast_to(x[0:1, :], (o, x.shape[1]))
        if o == n:
            return edge
        return jnp.concatenate([edge, x[: n - o, :]], axis=0)
    o = min(-off, n)
    edge = jnp.broadcast_to(x[n - 1 : n, :], (o, x.shape[1]))
    if o == n:
        return edge
    return jnp.concatenate([x[o:, :], edge], axis=0)


def _upsample_vpu(pn, g, uw_ref):
    r = _H // pn
    g_exp = jnp.broadcast_to(g[:, None, :], (pn, r, _C)).reshape(_H, _C)
    gp = None
    for t in range(4):
        off = r // 2 - (t - 1) * r
        term = uw_ref[:, t][:, None] * _shift_clamp(g_exp, off)
        gp = term if gp is None else gp + term
    return gp


def _phi_update(si, g, uw_ref, w_ref, b_ref, rest_blk):
    if si != _NSC - 1:
        gp = _upsample_vpu(_PNS[si], g, uw_ref)
    else:
        gp = g
    gpb = gp.astype(jnp.bfloat16)
    zrow = jnp.zeros((1, _C), jnp.bfloat16)
    sd = jnp.concatenate([zrow, gpb[:-1, :]], axis=0)
    su = jnp.concatenate([gpb[1:, :], zrow], axis=0)
    mm = lambda x, w: lax.dot_general(
        x, w, (((1,), (0,)), ((), ())), preferred_element_type=jnp.float32)
    w = w_ref[...].astype(jnp.bfloat16)
    y2 = mm(sd, w[0]) + mm(gpb, w[1]) + mm(su, w[2])
    h = 0.5 * gp + 0.5 * (y2 + b_ref[0, :][None, :])
    return rest_blk - h


def _merged_kernel(si, pn, pn_next, bb, nrb, g_ref, uw_ref, w_ref, b_ref,
                   rest_ref, cbn_ref, rest_out, idx_out, ss_out):
    i = pl.program_id(0) if nrb > 1 else 0
    r_next = _H // pn_next
    ss = None
    ds_list = []
    for b in range(bb):
        rnew = _phi_update(si, g_ref[b], uw_ref, w_ref, b_ref, rest_ref[b])
        rest_out[b] = rnew
        if r_next > 1:
            ds_list.append(jnp.mean(rnew.reshape(pn_next, r_next, _C), axis=1))
        else:
            ds_list.append(rnew)
        ssq = jnp.sum(rnew * rnew)
        ss = ssq if ss is None else ss + ssq

    if nrb == 1:
        ss_out[0, 0] = ss
    else:
        @pl.when(i == 0)
        def _():
            ss_out[0, 0] = ss

        @pl.when(i > 0)
        def _():
            ss_out[0, 0] += ss

    qn = _normalize_rows(jnp.concatenate(ds_list, axis=0)
                         if bb > 1 else ds_list[0])
    cbs = 2048
    m = None
    a = None
    for k in range(_VOCAB // cbs):
        s = _score(qn, cbn_ref[k * cbs:(k + 1) * cbs, :])
        lmax = jnp.max(s, axis=1)
        larg = jnp.argmax(s, axis=1).astype(jnp.int32) + k * cbs
        if m is None:
            m, a = lmax, larg
        else:
            better = lmax > m
            m = jnp.where(better, lmax, m)
            a = jnp.where(better, larg, a)
    idx_out[0, 0, :] = a


def _merged(si, g, rest, cbn, uw, w3, bias):
    pn = _PNS[si]
    pn_next = _PNS[si + 1]
    bb = min(_B, max(1, 1024 // pn_next))
    nrb = _B // bb
    qrows = bb * pn_next

    body = functools.partial(_merged_kernel, si, pn, pn_next, bb, nrb)
    return pl.pallas_call(
        body,
        grid=(nrb,),
        in_specs=[
            pl.BlockSpec((bb, pn, _C), lambda i: (i, 0, 0)),
            pl.BlockSpec((_H, 4), lambda i: (0, 0)),
            pl.BlockSpec((3, _C, _C), lambda i: (0, 0, 0)),
            pl.BlockSpec((1, _C), lambda i: (0, 0)),
            pl.BlockSpec((bb, _H, _C), lambda i: (i, 0, 0)),
            pl.BlockSpec((_VOCAB, _C), lambda i: (0, 0)),
        ],
        out_specs=[
            pl.BlockSpec((bb, _H, _C), lambda i: (i, 0, 0)),
            pl.BlockSpec((1, 1, qrows), lambda i: (i, 0, 0)),
            pl.BlockSpec((1, 1), lambda i: (0, 0), memory_space=pltpu.SMEM),
        ],
        out_shape=[
            jax.ShapeDtypeStruct((_B, _H, _C), jnp.float32),
            jax.ShapeDtypeStruct((nrb, 1, qrows), jnp.int32),
            jax.ShapeDtypeStruct((1, 1), jnp.float32),
        ],
    )(g.reshape(_B, pn, _C), uw, w3, bias, rest, cbn)


def _last_kernel(g_ref, w_ref, b_ref, rest_ref, f_ref, ss_out, fhat_out):
    ss = None
    for b in range(_B):
        rnew = _phi_update(_NSC - 1, g_ref[b], None, w_ref, b_ref, rest_ref[b])
        fhat_out[b] = f_ref[b] - jnp.transpose(rnew)
        ssq = jnp.sum(rnew * rnew)
        ss = ssq if ss is None else ss + ssq
    ss_out[0, 0] = ss


def _update_last(g, rest, f_nat, w3, bias):
    pn = _PNS[-1]
    return pl.pallas_call(
        _last_kernel,
        in_specs=[
            pl.BlockSpec((_B, pn, _C), lambda: (0, 0, 0)),
            pl.BlockSpec((3, _C, _C), lambda: (0, 0, 0)),
            pl.BlockSpec((1, _C), lambda: (0, 0)),
            pl.BlockSpec((_B, _H, _C), lambda: (0, 0, 0)),
            pl.BlockSpec((_B, _C, _H), lambda: (0, 0, 0)),
        ],
        out_specs=[
            pl.BlockSpec((1, 1), lambda: (0, 0), memory_space=pltpu.SMEM),
            pl.BlockSpec((_B, _C, _H), lambda: (0, 0, 0)),
        ],
        out_shape=[
            jax.ShapeDtypeStruct((1, 1), jnp.float32),
            jax.ShapeDtypeStruct((_B, _C, _H), jnp.float32),
        ],
    )(g.reshape(_B, pn, _C), w3, bias, rest, f_nat)


def kernel(f_BChw, codebook, phi_w, phi_b):
    f_nat = f_BChw.reshape(_B, _C, _H)

    cbn, ds0, f_r = _prologue(codebook, f_nat)

    w3s, biases = [], []
    for si in range(_NSC):
        k = _phi_share(si)
        w3s.append(jnp.transpose(phi_w[k, :, :, :, 1], (2, 1, 0)))
        biases.append(phi_b[k].reshape(1, _C))

    rest = f_r
    idx = _argmax0(ds0.reshape(_B, _C), cbn).reshape(_B)
    ss_list = []
    for si in range(_NSC - 1):
        g = _sc_gather(codebook, idx)
        uw = jnp.asarray(_upsample_weights(_PNS[si], _H))
        rest, idx3, ss = _merged(si, g, rest, cbn, uw, w3s[si], biases[si])
        idx = idx3.reshape(_B * _PNS[si + 1])
        ss_list.append(ss[0, 0])

    g = _sc_gather(codebook, idx)
    ss9, fhat = _update_last(g, rest, f_nat, w3s[-1], biases[-1])
    ss_list.append(ss9[0, 0])

    numel = _B * _H * _C
    loss = (1.0 + _BETA) / _NSC * jnp.sum(jnp.stack(ss_list)) / numel
    f_hat_out = fhat.reshape(_B, _C, _H, 1)
    return (f_hat_out, loss)


# speedup vs baseline: 2.2524x; 1.0009x over previous
import functools

import jax
import jax.numpy as jnp
import numpy as np
from jax import lax
from jax.experimental import pallas as pl
from jax.experimental.pallas import tpu as pltpu
from jax.experimental.pallas import tpu_sc as plsc

_PNS = (1, 2, 4, 8, 16, 32, 64, 128, 256, 512)
_VOCAB = 8192
_C = 256
_B = 16
_H = 512
_BETA = 0.25
_SHARE = 4
_NSC = 10


def _cubic_w_np(t, a=-0.75):
    at = np.abs(t)
    w1 = (a + 2.0) * at ** 3 - (a + 3.0) * at ** 2 + 1.0
    w2 = a * at ** 3 - 5.0 * a * at ** 2 + 8.0 * a * at - 4.0 * a
    return np.where(at <= 1.0, w1, np.where(at < 2.0, w2, np.zeros_like(at)))


@functools.lru_cache(maxsize=None)
def _upsample_weights(pn: int, out_h: int) -> np.ndarray:
    scale = pn / out_h
    i = np.arange(out_h, dtype=np.float32)
    src = (i + 0.5) * scale - 0.5
    i0 = np.floor(src).astype(np.int32)
    w = np.zeros((out_h, 4), dtype=np.float32)
    for t in range(4):
        tap = i0 - 1 + t
        w[:, t] = _cubic_w_np((src - tap).astype(np.float32))
    return w


@functools.lru_cache(maxsize=None)
def _phi_share(si: int) -> int:
    ticks = np.linspace(1.0 / 3.0 / _SHARE, 1.0 - 1.0 / 3.0 / _SHARE, _SHARE)
    return int(np.argmin(np.abs(ticks - si / (_NSC - 1))))


def _code_tile(pn_next: int) -> int:
    return _VOCAB if pn_next <= 64 else 2048


def _prologue_kernel(cb_ref, f_ref, cbn_ref, ds_ref, fr_ref):
    cb = cb_ref[...]
    nrm = jnp.sqrt(jnp.sum(cb * cb, axis=1))
    cbn_ref[...] = cb / jnp.maximum(nrm, 1e-12)[:, None]
    fb = f_ref[0]
    ds_ref[0, 0, :] = jnp.mean(fb, axis=1)
    fr_ref[0] = jnp.transpose(fb)


def _prologue(codebook, f_nat):
    return pl.pallas_call(
        _prologue_kernel,
        grid=(_B,),
        in_specs=[
            pl.BlockSpec((_VOCAB // _B, _C), lambda i: (i, 0)),
            pl.BlockSpec((1, _C, _H), lambda i: (i, 0, 0)),
        ],
        out_specs=[
            pl.BlockSpec((_VOCAB // _B, _C), lambda i: (i, 0)),
            pl.BlockSpec((1, 1, _C), lambda i: (i, 0, 0)),
            pl.BlockSpec((1, _H, _C), lambda i: (i, 0, 0)),
        ],
        out_shape=[
            jax.ShapeDtypeStruct((_VOCAB, _C), jnp.float32),
            jax.ShapeDtypeStruct((_B, 1, _C), jnp.float32),
            jax.ShapeDtypeStruct((_B, _H, _C), jnp.float32),
        ],
    )(codebook, f_nat)


def _normalize_rows(q):
    return q / jnp.maximum(jnp.sqrt(jnp.sum(q * q, axis=1)), 1e-12)[:, None]


def _score(qn, cbn):
    return lax.dot_general(qn.astype(jnp.bfloat16), cbn.astype(jnp.bfloat16),
                           (((1,), (1,)), ((), ())),
                           preferred_element_type=jnp.float32)


def _argmax0_kernel(q_ref, cbn_ref, idx_ref):
    qn = _normalize_rows(q_ref[...])
    s = _score(qn, cbn_ref[...])
    idx_ref[0, 0, :] = jnp.argmax(s, axis=1).astype(jnp.int32)


def _argmax0(ds0, cbn):
    return pl.pallas_call(
        _argmax0_kernel,
        out_specs=pl.BlockSpec((1, 1, _B), lambda: (0, 0, 0)),
        out_shape=jax.ShapeDtypeStruct((1, 1, _B), jnp.int32),
    )(ds0, cbn)


def _sc_gather(codebook, idx):
    info = plsc.get_sparse_core_info()
    nw = info.num_cores * info.num_subcores
    n = idx.shape[0]
    nw_used = min(nw, n // 8)
    b_per_w = n // nw_used
    mesh = plsc.VectorSubcoreMesh(core_axis_name="c", subcore_axis_name="s")

    @functools.partial(
        pl.kernel, mesh=mesh,
        out_type=jax.ShapeDtypeStruct((n, _C), jnp.float32),
        scratch_types=[
            pltpu.VMEM((b_per_w,), jnp.int32),
            pltpu.VMEM((b_per_w, _C), jnp.float32),
            pltpu.SemaphoreType.DMA,
        ],
    )
    def k(table_hbm, idx_hbm, out_hbm, idx_v, rows_v, sem):
        wid = lax.axis_index("s") * info.num_cores + lax.axis_index("c")

        @pl.when(wid < nw_used)
        def _():
            base = wid * b_per_w
            pltpu.sync_copy(idx_hbm.at[pl.ds(base, b_per_w)], idx_v)
            pltpu.async_copy(table_hbm.at[idx_v], rows_v, sem).wait()
            pltpu.sync_copy(rows_v, out_hbm.at[pl.ds(base, b_per_w)])

    return k(codebook, idx)


def _shift_clamp(x, off):
    n = x.shape[0]
    if off == 0:
        return x
    if off > 0:
        o = min(off, n)
        edge = jnp.broadcast_to(x[0:1, :], (o, x.shape[1]))
        if o == n:
            return edge
        return jnp.concatenate([edge, x[: n - o, :]], axis=0)
    o = min(-off, n)
    edge = jnp.broadcast_to(x[n - 1 : n, :], (o, x.shape[1]))
    if o == n:
        return edge
    return jnp.concatenate([x[o:, :], edge], axis=0)


def _upsample_vpu(pn, g, uw_ref):
    r = _H // pn
    g_exp = jnp.broadcast_to(g[:, None, :], (pn, r, _C)).reshape(_H, _C)
    gp = None
    for t in range(4):
        off = r // 2 - (t - 1) * r
        term = uw_ref[:, t][:, None] * _shift_clamp(g_exp, off)
        gp = term if gp is None else gp + term
    return gp


def _phi_update(si, g, uw_ref, w_ref, b_ref, rest_blk):
    if si != _NSC - 1:
        gp = _upsample_vpu(_PNS[si], g, uw_ref)
    else:
        gp = g
    gpb = gp.astype(jnp.bfloat16)
    zrow = jnp.zeros((1, _C), jnp.bfloat16)
    sd = jnp.concatenate([zrow, gpb[:-1, :]], axis=0)
    su = jnp.concatenate([gpb[1:, :], zrow], axis=0)
    mm = lambda x, w: lax.dot_general(
        x, w, (((1,), (0,)), ((), ())), preferred_element_type=jnp.float32)
    w = w_ref[...].astype(jnp.bfloat16)
    y2 = mm(sd, w[0]) + mm(gpb, w[1]) + mm(su, w[2])
    h = 0.5 * gp + 0.5 * (y2 + b_ref[0, :][None, :])
    return rest_blk - h


def _merged_kernel(si, pn, pn_next, bb, nrb, g_ref, uw_ref, w_ref, b_ref,
                   rest_ref, cbn_ref, rest_out, idx_out, ss_out):
    i = pl.program_id(0) if nrb > 1 else 0
    r_next = _H // pn_next
    ss = None
    ds_list = []
    for b in range(bb):
        rnew = _phi_update(si, g_ref[b], uw_ref, w_ref, b_ref, rest_ref[b])
        rest_out[b] = rnew
        if r_next > 1:
            ds_list.append(jnp.mean(rnew.reshape(pn_next, r_next, _C), axis=1))
        else:
            ds_list.append(rnew)
        ssq = jnp.sum(rnew * rnew)
        ss = ssq if ss is None else ss + ssq

    if nrb == 1:
        ss_out[0, 0] = ss
    else:
        @pl.when(i == 0)
        def _():
            ss_out[0, 0] = ss

        @pl.when(i > 0)
        def _():
            ss_out[0, 0] += ss

    qn = _normalize_rows(jnp.concatenate(ds_list, axis=0)
                         if bb > 1 else ds_list[0])
    cbs = 2048
    m = None
    a = None
    for k in range(_VOCAB // cbs):
        s = _score(qn, cbn_ref[k * cbs:(k + 1) * cbs, :])
        lmax = jnp.max(s, axis=1)
        larg = jnp.argmax(s, axis=1).astype(jnp.int32) + k * cbs
        if m is None:
            m, a = lmax, larg
        else:
            better = lmax > m
            m = jnp.where(better, lmax, m)
            a = jnp.where(better, larg, a)
    idx_out[0, 0, :] = a


def _merged(si, g, rest, cbn, uw, w3, bias):
    pn = _PNS[si]
    pn_next = _PNS[si + 1]
    bb = min(_B, max(1, 1024 // pn_next))
    nrb = _B // bb
    qrows = bb * pn_next

    body = functools.partial(_merged_kernel, si, pn, pn_next, bb, nrb)
    return pl.pallas_call(
        body,
        grid=(nrb,),
        in_specs=[
            pl.BlockSpec((bb, pn, _C), lambda i: (i, 0, 0)),
            pl.BlockSpec((_H, 4), lambda i: (0, 0)),
            pl.BlockSpec((3, _C, _C), lambda i: (0, 0, 0)),
            pl.BlockSpec((1, _C), lambda i: (0, 0)),
            pl.BlockSpec((bb, _H, _C), lambda i: (i, 0, 0)),
            pl.BlockSpec((_VOCAB, _C), lambda i: (0, 0)),
        ],
        out_specs=[
            pl.BlockSpec((bb, _H, _C), lambda i: (i, 0, 0)),
            pl.BlockSpec((1, 1, qrows), lambda i: (i, 0, 0)),
            pl.BlockSpec((1, 1), lambda i: (0, 0), memory_space=pltpu.SMEM),
        ],
        out_shape=[
            jax.ShapeDtypeStruct((_B, _H, _C), jnp.float32),
            jax.ShapeDtypeStruct((nrb, 1, qrows), jnp.int32),
            jax.ShapeDtypeStruct((1, 1), jnp.float32),
        ],
    )(g.reshape(_B, pn, _C), uw, w3, bias, rest, cbn)


def _last_kernel(g_ref, w_ref, b_ref, rest_ref, f_ref, ss_out, fhat_out):
    ss = None
    for b in range(_B):
        rnew = _phi_update(_NSC - 1, g_ref[b], None, w_ref, b_ref, rest_ref[b])
        fhat_out[b] = f_ref[b] - jnp.transpose(rnew)
        ssq = jnp.sum(rnew * rnew)
        ss = ssq if ss is None else ss + ssq
    ss_out[0, 0] = ss


def _update_last(g, rest, f_nat, w3, bias):
    pn = _PNS[-1]
    return pl.pallas_call(
        _last_kernel,
        in_specs=[
            pl.BlockSpec((_B, pn, _C), lambda: (0, 0, 0)),
            pl.BlockSpec((3, _C, _C), lambda: (0, 0, 0)),
            pl.BlockSpec((1, _C), lambda: (0, 0)),
            pl.BlockSpec((_B, _H, _C), lambda: (0, 0, 0)),
            pl.BlockSpec((_B, _C, _H), lambda: (0, 0, 0)),
        ],
        out_specs=[
            pl.BlockSpec((1, 1), lambda: (0, 0), memory_space=pltpu.SMEM),
            pl.BlockSpec((_B, _C, _H), lambda: (0, 0, 0)),
        ],
        out_shape=[
            jax.ShapeDtypeStruct((1, 1), jnp.float32),
            jax.ShapeDtypeStruct((_B, _C, _H), jnp.float32),
        ],
    )(g.reshape(_B, pn, _C), w3, bias, rest, f_nat)


def kernel(f_BChw, codebook, phi_w, phi_b):
    f_nat = f_BChw.reshape(_B, _C, _H)

    cbn, ds0, f_r = _prologue(codebook, f_nat)

    w3s, biases = [], []
    for si in range(_NSC):
        k = _phi_share(si)
        w3s.append(jnp.transpose(phi_w[k, :, :, :, 1], (2, 1, 0)))
        biases.append(phi_b[k].reshape(1, _C))

    rest = f_r
    idx = _argmax0(ds0.reshape(_B, _C), cbn).reshape(_B)
    ss_list = []
    for si in range(_NSC - 1):
        g = _sc_gather(codebook, idx)
        uw = jnp.asarray(_upsample_weights(_PNS[si], _H))
        rest, idx3, ss = _merged(si, g, rest, cbn, uw, w3s[si], biases[si])
        idx = idx3.reshape(_B * _PNS[si + 1])
        ss_list.append(ss[0, 0])

    g = _sc_gather(codebook, idx)
    ss9, fhat = _update_last(g, rest, f_nat, w3s[-1], biases[-1])
    ss_list.append(ss9[0, 0])

    numel = _B * _H * _C
    loss = (1.0 + _BETA) / _NSC * jnp.sum(jnp.stack(ss_list)) / numel
    f_hat_out = fhat.reshape(_B, _C, _H, 1)
    return (f_hat_out, loss)
